# baseline (speedup 1.0000x reference)
import functools

import jax
import jax.numpy as jnp
from jax import lax
from jax.experimental import pallas as pl
from jax.experimental.pallas import tpu as pltpu
from jax.experimental.pallas import tpu_sc as plsc

_N = 10000
_E = 320000
_D = 128
_H = 128
_HH = 64
_NOTES = 128

_NC = 2
_NS = 16
_B = 128
_NB = 160
_EPAD = _NS * _NB * _B
_ACC_R = 10240
_SR = _ACC_R // _NS
_K = 5
_S = 3
_G = _K - _S
_NBLK = _NB // _K


def _segsum_body(h_hbm, src_hbm, dst_hbm, zeros_hbm, out_hbm,
                 srcb, dstb, acc, hst, *rest):
    c = lax.axis_index("c")
    s = lax.axis_index("s")
    bufs = rest[:_K]
    gsem = rest[_K:2 * _K]
    ssem = rest[2 * _K:3 * _K]
    isem0, isem1, zsem, hsem = rest[3 * _K:]
    hme = hst

    hoff = jnp.minimum(s * _SR, _N - _SR)
    pltpu.async_copy(zeros_hbm.at[pl.ds(s * _SR, _SR)],
                     acc.at[pl.ds(s * _SR, _SR)], zsem)
    pltpu.async_copy(h_hbm.at[c, pl.ds(hoff, _SR)],
                     hst.at[pl.ds(hoff, _SR)], hsem)
    pltpu.async_copy(src_hbm.at[s, pl.ds(0, _K)], srcb.at[0], isem0)
    pltpu.async_copy(dst_hbm.at[s, pl.ds(0, _K)], dstb.at[0], isem1)
    pltpu.make_async_copy(zeros_hbm.at[pl.ds(s * _SR, _SR)],
                          acc.at[pl.ds(s * _SR, _SR)], zsem).wait()
    pltpu.make_async_copy(h_hbm.at[c, pl.ds(hoff, _SR)],
                          hst.at[pl.ds(hoff, _SR)], hsem).wait()
    pltpu.make_async_copy(src_hbm.at[s, pl.ds(0, _K)], srcb.at[0],
                          isem0).wait()
    pltpu.make_async_copy(dst_hbm.at[s, pl.ds(0, _K)], dstb.at[0],
                          isem1).wait()
    plsc.subcore_barrier()

    def fire_idx(blk, slot):
        pltpu.async_copy(src_hbm.at[s, pl.ds(blk * _K, _K)],
                         srcb.at[slot], isem0)
        pltpu.async_copy(dst_hbm.at[s, pl.ds(blk * _K, _K)],
                         dstb.at[slot], isem1)

    def wait_idx(blk, slot):
        pltpu.make_async_copy(src_hbm.at[s, pl.ds(blk * _K, _K)],
                              srcb.at[slot], isem0).wait()
        pltpu.make_async_copy(dst_hbm.at[s, pl.ds(blk * _K, _K)],
                              dstb.at[slot], isem1).wait()

    def fire_gather(slot, row, k):
        pltpu.async_copy(hme.at[srcb.at[slot, row]], bufs[k], gsem[k])

    def wait_gather(slot, row, k):
        pltpu.make_async_copy(hme.at[srcb.at[slot, row]], bufs[k],
                              gsem[k]).wait()

    def fire_scatter(slot, row, k):
        pltpu.async_copy(bufs[k], acc.at[dstb.at[slot, row]], ssem[k],
                         add=True)

    def wait_scatter(slot, row, k):
        pltpu.make_async_copy(bufs[k], acc.at[dstb.at[slot, row]],
                              ssem[k]).wait()

    def block(i, slot, first, last):
        nslot = 1 - slot
        for k in range(_K):
            if not last:
                if k == 0:
                    fire_idx(i + 1, nslot)
                if k == _S:
                    wait_idx(i + 1, nslot)
            if k < _S:
                if not first:
                    wait_scatter(nslot, k - _S + _K, (k - _S) % _K)
            else:
                wait_scatter(slot, k - _S, (k - _S) % _K)
            if k < _S:
                fire_gather(slot, k + _G, (k + _G) % _K)
            elif not last:
                fire_gather(nslot, k + _G - _K, (k + _G) % _K)
            wait_gather(slot, k, k)
            fire_scatter(slot, k, k)

    for k in range(_G):
        fire_gather(0, k, k)
    block(0, 0, True, False)

    def pair(bi, carry):
        block(2 * bi + 1, 1, False, False)
        block(2 * bi + 2, 0, False, False)
        return carry

    lax.fori_loop(0, (_NBLK - 2) // 2, pair, 0)

    block(_NBLK - 1, 1, False, True)
    for k in range(_S):
        wait_scatter(1, _G + k, (_G + k) % _K)

    plsc.subcore_barrier()
    pltpu.sync_copy(acc.at[pl.ds(s * _SR, _SR)],
                    out_hbm.at[c, pl.ds(s * _SR, _SR)])


_segsum = pl.kernel(
    _segsum_body,
    mesh=plsc.VectorSubcoreMesh(core_axis_name="c", subcore_axis_name="s"),
    compiler_params=pltpu.CompilerParams(use_tc_tiling_on_sc=False),
    out_type=jax.ShapeDtypeStruct((_NC, _ACC_R, _HH), jnp.float32),
    scratch_types=(
        [pltpu.VMEM((2, _K, _B), jnp.int32),
         pltpu.VMEM((2, _K, _B), jnp.int32),
         pltpu.VMEM_SHARED((_ACC_R, _HH), jnp.float32),
         pltpu.VMEM_SHARED((_ACC_R, _HH), jnp.float32)]
        + [pltpu.VMEM((_B, _HH), jnp.float32) for _ in range(_K)]
        + [pltpu.SemaphoreType.DMA for _ in range(2 * _K + 4)]),
)


def _pre_body(x_ref, g_ref, b_ref, w_ref, pb_ref, o_ref):
    x = x_ref[...]
    m = jnp.mean(x, axis=0, keepdims=True)
    d = x - m
    v = jnp.mean(d * d, axis=0, keepdims=True)
    hn = g_ref[...] * d * lax.rsqrt(v + 1e-5) + b_ref[...]
    w = w_ref[...]
    pb = pb_ref[...]
    o_ref[0] = jnp.maximum(
        jnp.dot(hn, w[:, :_HH], preferred_element_type=jnp.float32)
        + pb[:, :_HH], 0.0)
    o_ref[1] = jnp.maximum(
        jnp.dot(hn, w[:, _HH:], preferred_element_type=jnp.float32)
        + pb[:, _HH:], 0.0)


_pre = pl.pallas_call(
    _pre_body,
    out_shape=jax.ShapeDtypeStruct((2, _N, _HH), jnp.float32),
)


def _half_mlp_bn(z2, g, be):
    m = jnp.mean(z2, axis=0, keepdims=True)
    d = z2 - m
    v = jnp.mean(d * d, axis=0, keepdims=True)
    return g * d * lax.rsqrt(v + 1e-5) + be


def _conv_body(h_ref, a_ref, w1_ref, b1_ref, w2_ref, b2_ref,
               g_ref, be_ref, *rest, add_res):
    if add_res:
        r_ref, o_ref = rest
    else:
        (o_ref,) = rest
    z0 = h_ref[0] + a_ref[0, :_N]
    z1 = h_ref[1] + a_ref[1, :_N]
    w1 = w1_ref[...]
    u = jnp.maximum(
        jnp.dot(z0, w1[:_HH, :], preferred_element_type=jnp.float32)
        + jnp.dot(z1, w1[_HH:, :], preferred_element_type=jnp.float32)
        + b1_ref[...], 0.0)
    w2 = w2_ref[...]
    b2 = b2_ref[...]
    g = g_ref[...]
    be = be_ref[...]
    z2_0 = jnp.dot(u, w2[:, :_HH], preferred_element_type=jnp.float32) \
        + b2[:, :_HH]
    z2_1 = jnp.dot(u, w2[:, _HH:], preferred_element_type=jnp.float32) \
        + b2[:, _HH:]
    h2_0 = jnp.maximum(_half_mlp_bn(z2_0, g[:, :_HH], be[:, :_HH]), 0.0)
    h2_1 = jnp.maximum(_half_mlp_bn(z2_1, g[:, _HH:], be[:, _HH:]), 0.0)
    if add_res:
        h2_0 = h2_0 + r_ref[0]
        h2_1 = h2_1 + r_ref[1]
    o_ref[0] = h2_0
    o_ref[1] = h2_1


_conv = pl.pallas_call(
    functools.partial(_conv_body, add_res=False),
    out_shape=jax.ShapeDtypeStruct((2, _N, _HH), jnp.float32),
)

_conv_res = pl.pallas_call(
    functools.partial(_conv_body, add_res=True),
    out_shape=jax.ShapeDtypeStruct((2, _N, _HH), jnp.float32),
)


def _conv_post_body(h_ref, a_ref, w1_ref, b1_ref, w2_ref, b2_ref,
                    g_ref, be_ref, r_ref, w_ref, b_ref, emb_ref, log_ref):
    z0 = h_ref[0] + a_ref[0, :_N]
    z1 = h_ref[1] + a_ref[1, :_N]
    w1 = w1_ref[...]
    u = jnp.maximum(
        jnp.dot(z0, w1[:_HH, :], preferred_element_type=jnp.float32)
        + jnp.dot(z1, w1[_HH:, :], preferred_element_type=jnp.float32)
        + b1_ref[...], 0.0)
    w2 = w2_ref[...]
    b2 = b2_ref[...]
    g = g_ref[...]
    be = be_ref[...]
    z2_0 = jnp.dot(u, w2[:, :_HH], preferred_element_type=jnp.float32) \
        + b2[:, :_HH]
    z2_1 = jnp.dot(u, w2[:, _HH:], preferred_element_type=jnp.float32) \
        + b2[:, _HH:]
    h2_0 = jnp.maximum(_half_mlp_bn(z2_0, g[:, :_HH], be[:, :_HH]), 0.0) \
        + r_ref[0]
    h2_1 = jnp.maximum(_half_mlp_bn(z2_1, g[:, _HH:], be[:, _HH:]), 0.0) \
        + r_ref[1]
    p0 = jnp.mean(h2_0, axis=0, keepdims=True)
    p1 = jnp.mean(h2_1, axis=0, keepdims=True)
    emb_ref[:, :_HH] = p0
    emb_ref[:, _HH:] = p1
    w = w_ref[...]
    log_ref[...] = (
        jnp.dot(p0, w[:_HH, :], preferred_element_type=jnp.float32)
        + jnp.dot(p1, w[_HH:, :], preferred_element_type=jnp.float32)
        + b_ref[...])


_conv_post = pl.pallas_call(
    _conv_post_body,
    out_shape=(jax.ShapeDtypeStruct((1, _H), jnp.float32),
               jax.ShapeDtypeStruct((1, _NOTES), jnp.float32)),
)


def kernel(x, edge_index, fn_g, fn_b, proj_W, proj_b,
           W1_0, b1_0, W2_0, b2_0, g_0, be_0,
           W1_1, b1_1, W2_1, b2_1, g_1, be_1,
           W1_2, b1_2, W2_2, b2_2, g_2, be_2,
           pred_W, pred_b):
    src = edge_index[0]
    dst = edge_index[1]
    pad = _EPAD - _E
    src_p = jnp.concatenate(
        [src, jnp.zeros((pad,), jnp.int32)]).reshape(_NS, _NB, _B)
    dst_p = jnp.concatenate(
        [dst, jnp.full((pad,), _N, jnp.int32)]).reshape(_NS, _NB, _B)
    zeros = jnp.zeros((_ACC_R, _HH), jnp.float32)

    r2 = lambda t: t.reshape(1, -1)
    h = _pre(x, r2(fn_g), r2(fn_b), proj_W, r2(proj_b))
    res = h
    mlps = [(W1_0, r2(b1_0), W2_0, r2(b2_0), r2(g_0), r2(be_0)),
            (W1_1, r2(b1_1), W2_1, r2(b2_1), r2(g_1), r2(be_1)),
            (W1_2, r2(b1_2), W2_2, r2(b2_2), r2(g_2), r2(be_2))]
    for mi, (W1, b1, W2, b2, g, be) in enumerate(mlps):
        a = _segsum(h, src_p, dst_p, zeros)
        h = _conv(h, a, W1, b1, W2, b2, g, be)
        a = _segsum(h, src_p, dst_p, zeros)
        if mi < 2:
            h = _conv_res(h, a, W1, b1, W2, b2, g, be, res)
            res = h
        else:
            embed, logits = _conv_post(h, a, W1, b1, W2, b2, g, be, res,
                                       pred_W, r2(pred_b))
    return (embed, logits)


# speedup vs baseline: 1.0011x; 1.0011x over previous
import functools

import jax
import jax.numpy as jnp
from jax import lax
from jax.experimental import pallas as pl
from jax.experimental.pallas import tpu as pltpu
from jax.experimental.pallas import tpu_sc as plsc

_N = 10000
_E = 320000
_D = 128
_H = 128
_HH = 64
_NOTES = 128

_NC = 2
_NS = 16
_B = 128
_NB = 160
_EPAD = _NS * _NB * _B
_ACC_R = 10240
_SR = _ACC_R // _NS
_K = 5
_S = 2
_G = _K - _S
_NBLK = _NB // _K


def _segsum_body(h_hbm, src_hbm, dst_hbm, zeros_hbm, out_hbm,
                 srcb, dstb, acc, hst, *rest):
    c = lax.axis_index("c")
    s = lax.axis_index("s")
    bufs = rest[:_K]
    gsem = rest[_K:2 * _K]
    ssem = rest[2 * _K:3 * _K]
    isem0, isem1, zsem, hsem = rest[3 * _K:]
    hme = hst

    hoff = jnp.minimum(s * _SR, _N - _SR)
    pltpu.async_copy(zeros_hbm.at[pl.ds(s * _SR, _SR)],
                     acc.at[pl.ds(s * _SR, _SR)], zsem)
    pltpu.async_copy(h_hbm.at[c, pl.ds(hoff, _SR)],
                     hst.at[pl.ds(hoff, _SR)], hsem)
    pltpu.async_copy(src_hbm.at[s, pl.ds(0, _K)], srcb.at[0], isem0)
    pltpu.async_copy(dst_hbm.at[s, pl.ds(0, _K)], dstb.at[0], isem1)
    pltpu.make_async_copy(zeros_hbm.at[pl.ds(s * _SR, _SR)],
                          acc.at[pl.ds(s * _SR, _SR)], zsem).wait()
    pltpu.make_async_copy(h_hbm.at[c, pl.ds(hoff, _SR)],
                          hst.at[pl.ds(hoff, _SR)], hsem).wait()
    pltpu.make_async_copy(src_hbm.at[s, pl.ds(0, _K)], srcb.at[0],
                          isem0).wait()
    pltpu.make_async_copy(dst_hbm.at[s, pl.ds(0, _K)], dstb.at[0],
                          isem1).wait()
    plsc.subcore_barrier()

    def fire_idx(blk, slot):
        pltpu.async_copy(src_hbm.at[s, pl.ds(blk * _K, _K)],
                         srcb.at[slot], isem0)
        pltpu.async_copy(dst_hbm.at[s, pl.ds(blk * _K, _K)],
                         dstb.at[slot], isem1)

    def wait_idx(blk, slot):
        pltpu.make_async_copy(src_hbm.at[s, pl.ds(blk * _K, _K)],
                              srcb.at[slot], isem0).wait()
        pltpu.make_async_copy(dst_hbm.at[s, pl.ds(blk * _K, _K)],
                              dstb.at[slot], isem1).wait()

    def fire_gather(slot, row, k):
        pltpu.async_copy(hme.at[srcb.at[slot, row]], bufs[k], gsem[k])

    def wait_gather(slot, row, k):
        pltpu.make_async_copy(hme.at[srcb.at[slot, row]], bufs[k],
                              gsem[k]).wait()

    def fire_scatter(slot, row, k):
        pltpu.async_copy(bufs[k], acc.at[dstb.at[slot, row]], ssem[k],
                         add=True)

    def wait_scatter(slot, row, k):
        pltpu.make_async_copy(bufs[k], acc.at[dstb.at[slot, row]],
                              ssem[k]).wait()

    def block(i, slot, first, last):
        nslot = 1 - slot
        for k in range(_K):
            if not last:
                if k == 0:
                    fire_idx(i + 1, nslot)
                if k == _S:
                    wait_idx(i + 1, nslot)
            if k < _S:
                if not first:
                    wait_scatter(nslot, k - _S + _K, (k - _S) % _K)
            else:
                wait_scatter(slot, k - _S, (k - _S) % _K)
            if k < _S:
                fire_gather(slot, k + _G, (k + _G) % _K)
            elif not last:
                fire_gather(nslot, k + _G - _K, (k + _G) % _K)
            wait_gather(slot, k, k)
            fire_scatter(slot, k, k)

    for k in range(_G):
        fire_gather(0, k, k)
    block(0, 0, True, False)

    def pair(bi, carry):
        block(2 * bi + 1, 1, False, False)
        block(2 * bi + 2, 0, False, False)
        return carry

    lax.fori_loop(0, (_NBLK - 2) // 2, pair, 0)

    block(_NBLK - 1, 1, False, True)
    for k in range(_S):
        wait_scatter(1, _G + k, (_G + k) % _K)

    plsc.subcore_barrier()
    pltpu.sync_copy(acc.at[pl.ds(s * _SR, _SR)],
                    out_hbm.at[c, pl.ds(s * _SR, _SR)])


_segsum = pl.kernel(
    _segsum_body,
    mesh=plsc.VectorSubcoreMesh(core_axis_name="c", subcore_axis_name="s"),
    compiler_params=pltpu.CompilerParams(use_tc_tiling_on_sc=False),
    out_type=jax.ShapeDtypeStruct((_NC, _ACC_R, _HH), jnp.float32),
    scratch_types=(
        [pltpu.VMEM((2, _K, _B), jnp.int32),
         pltpu.VMEM((2, _K, _B), jnp.int32),
         pltpu.VMEM_SHARED((_ACC_R, _HH), jnp.float32),
         pltpu.VMEM_SHARED((_ACC_R, _HH), jnp.float32)]
        + [pltpu.VMEM((_B, _HH), jnp.float32) for _ in range(_K)]
        + [pltpu.SemaphoreType.DMA for _ in range(2 * _K + 4)]),
)


def _pre_body(x_ref, g_ref, b_ref, w_ref, pb_ref, o_ref):
    x = x_ref[...]
    m = jnp.mean(x, axis=0, keepdims=True)
    d = x - m
    v = jnp.mean(d * d, axis=0, keepdims=True)
    hn = g_ref[...] * d * lax.rsqrt(v + 1e-5) + b_ref[...]
    w = w_ref[...]
    pb = pb_ref[...]
    o_ref[0] = jnp.maximum(
        jnp.dot(hn, w[:, :_HH], preferred_element_type=jnp.float32)
        + pb[:, :_HH], 0.0)
    o_ref[1] = jnp.maximum(
        jnp.dot(hn, w[:, _HH:], preferred_element_type=jnp.float32)
        + pb[:, _HH:], 0.0)


_pre = pl.pallas_call(
    _pre_body,
    out_shape=jax.ShapeDtypeStruct((2, _N, _HH), jnp.float32),
)


def _half_mlp_bn(z2, g, be):
    m = jnp.mean(z2, axis=0, keepdims=True)
    d = z2 - m
    v = jnp.mean(d * d, axis=0, keepdims=True)
    return g * d * lax.rsqrt(v + 1e-5) + be


def _conv_body(h_ref, a_ref, w1_ref, b1_ref, w2_ref, b2_ref,
               g_ref, be_ref, *rest, add_res):
    if add_res:
        r_ref, o_ref = rest
    else:
        (o_ref,) = rest
    z0 = h_ref[0] + a_ref[0, :_N]
    z1 = h_ref[1] + a_ref[1, :_N]
    w1 = w1_ref[...]
    u = jnp.maximum(
        jnp.dot(z0, w1[:_HH, :], preferred_element_type=jnp.float32)
        + jnp.dot(z1, w1[_HH:, :], preferred_element_type=jnp.float32)
        + b1_ref[...], 0.0)
    w2 = w2_ref[...]
    b2 = b2_ref[...]
    g = g_ref[...]
    be = be_ref[...]
    z2_0 = jnp.dot(u, w2[:, :_HH], preferred_element_type=jnp.float32) \
        + b2[:, :_HH]
    z2_1 = jnp.dot(u, w2[:, _HH:], preferred_element_type=jnp.float32) \
        + b2[:, _HH:]
    h2_0 = jnp.maximum(_half_mlp_bn(z2_0, g[:, :_HH], be[:, :_HH]), 0.0)
    h2_1 = jnp.maximum(_half_mlp_bn(z2_1, g[:, _HH:], be[:, _HH:]), 0.0)
    if add_res:
        h2_0 = h2_0 + r_ref[0]
        h2_1 = h2_1 + r_ref[1]
    o_ref[0] = h2_0
    o_ref[1] = h2_1


_conv = pl.pallas_call(
    functools.partial(_conv_body, add_res=False),
    out_shape=jax.ShapeDtypeStruct((2, _N, _HH), jnp.float32),
)

_conv_res = pl.pallas_call(
    functools.partial(_conv_body, add_res=True),
    out_shape=jax.ShapeDtypeStruct((2, _N, _HH), jnp.float32),
)


def _conv_post_body(h_ref, a_ref, w1_ref, b1_ref, w2_ref, b2_ref,
                    g_ref, be_ref, r_ref, w_ref, b_ref, emb_ref, log_ref):
    z0 = h_ref[0] + a_ref[0, :_N]
    z1 = h_ref[1] + a_ref[1, :_N]
    w1 = w1_ref[...]
    u = jnp.maximum(
        jnp.dot(z0, w1[:_HH, :], preferred_element_type=jnp.float32)
        + jnp.dot(z1, w1[_HH:, :], preferred_element_type=jnp.float32)
        + b1_ref[...], 0.0)
    w2 = w2_ref[...]
    b2 = b2_ref[...]
    g = g_ref[...]
    be = be_ref[...]
    z2_0 = jnp.dot(u, w2[:, :_HH], preferred_element_type=jnp.float32) \
        + b2[:, :_HH]
    z2_1 = jnp.dot(u, w2[:, _HH:], preferred_element_type=jnp.float32) \
        + b2[:, _HH:]
    h2_0 = jnp.maximum(_half_mlp_bn(z2_0, g[:, :_HH], be[:, :_HH]), 0.0) \
        + r_ref[0]
    h2_1 = jnp.maximum(_half_mlp_bn(z2_1, g[:, _HH:], be[:, _HH:]), 0.0) \
        + r_ref[1]
    p0 = jnp.mean(h2_0, axis=0, keepdims=True)
    p1 = jnp.mean(h2_1, axis=0, keepdims=True)
    emb_ref[:, :_HH] = p0
    emb_ref[:, _HH:] = p1
    w = w_ref[...]
    log_ref[...] = (
        jnp.dot(p0, w[:_HH, :], preferred_element_type=jnp.float32)
        + jnp.dot(p1, w[_HH:, :], preferred_element_type=jnp.float32)
        + b_ref[...])


_conv_post = pl.pallas_call(
    _conv_post_body,
    out_shape=(jax.ShapeDtypeStruct((1, _H), jnp.float32),
               jax.ShapeDtypeStruct((1, _NOTES), jnp.float32)),
)


def kernel(x, edge_index, fn_g, fn_b, proj_W, proj_b,
           W1_0, b1_0, W2_0, b2_0, g_0, be_0,
           W1_1, b1_1, W2_1, b2_1, g_1, be_1,
           W1_2, b1_2, W2_2, b2_2, g_2, be_2,
           pred_W, pred_b):
    src = edge_index[0]
    dst = edge_index[1]
    pad = _EPAD - _E
    src_p = jnp.concatenate(
        [src, jnp.zeros((pad,), jnp.int32)]).reshape(_NS, _NB, _B)
    dst_p = jnp.concatenate(
        [dst, jnp.full((pad,), _N, jnp.int32)]).reshape(_NS, _NB, _B)
    zeros = jnp.zeros((_ACC_R, _HH), jnp.float32)

    r2 = lambda t: t.reshape(1, -1)
    h = _pre(x, r2(fn_g), r2(fn_b), proj_W, r2(proj_b))
    res = h
    mlps = [(W1_0, r2(b1_0), W2_0, r2(b2_0), r2(g_0), r2(be_0)),
            (W1_1, r2(b1_1), W2_1, r2(b2_1), r2(g_1), r2(be_1)),
            (W1_2, r2(b1_2), W2_2, r2(b2_2), r2(g_2), r2(be_2))]
    for mi, (W1, b1, W2, b2, g, be) in enumerate(mlps):
        a = _segsum(h, src_p, dst_p, zeros)
        h = _conv(h, a, W1, b1, W2, b2, g, be)
        a = _segsum(h, src_p, dst_p, zeros)
        if mi < 2:
            h = _conv_res(h, a, W1, b1, W2, b2, g, be, res)
            res = h
        else:
            embed, logits = _conv_post(h, a, W1, b1, W2, b2, g, be, res,
                                       pred_W, r2(pred_b))
    return (embed, logits)


# speedup vs baseline: 1.0296x; 1.0285x over previous
import functools

import jax
import jax.numpy as jnp
from jax import lax
from jax.experimental import pallas as pl
from jax.experimental.pallas import tpu as pltpu
from jax.experimental.pallas import tpu_sc as plsc

_N = 10000
_E = 320000
_D = 128
_H = 128
_HH = 64
_NOTES = 128

_NC = 2
_NS = 16
_B = 128
_NB = 160
_EPAD = _NS * _NB * _B
_ACC_R = 10240
_SR = _ACC_R // _NS
_K = 5
_S = 2
_G = _K - _S
_NBLK = _NB // _K


def _segsum_body(h_hbm, src_hbm, dst_hbm, zeros_hbm, out_hbm,
                 srcb, dstb, acc, hst, *rest):
    c = lax.axis_index("c")
    s = lax.axis_index("s")
    bufs = rest[:_K]
    gsem = rest[_K:2 * _K]
    ssem = rest[2 * _K:3 * _K]
    isem0, isem1, zsem, hsem = rest[3 * _K:]
    hme = hst

    hoff = jnp.minimum(s * _SR, _N - _SR)
    pltpu.async_copy(zeros_hbm.at[pl.ds(s * _SR, _SR)],
                     acc.at[pl.ds(s * _SR, _SR)], zsem)
    pltpu.async_copy(h_hbm.at[c, pl.ds(hoff, _SR)],
                     hst.at[pl.ds(hoff, _SR)], hsem)
    pltpu.async_copy(src_hbm.at[s, pl.ds(0, _K)], srcb.at[0], isem0)
    pltpu.async_copy(dst_hbm.at[s, pl.ds(0, _K)], dstb.at[0], isem1)
    pltpu.make_async_copy(zeros_hbm.at[pl.ds(s * _SR, _SR)],
                          acc.at[pl.ds(s * _SR, _SR)], zsem).wait()
    pltpu.make_async_copy(h_hbm.at[c, pl.ds(hoff, _SR)],
                          hst.at[pl.ds(hoff, _SR)], hsem).wait()
    pltpu.make_async_copy(src_hbm.at[s, pl.ds(0, _K)], srcb.at[0],
                          isem0).wait()
    pltpu.make_async_copy(dst_hbm.at[s, pl.ds(0, _K)], dstb.at[0],
                          isem1).wait()
    plsc.subcore_barrier()

    def fire_idx(blk, slot):
        pltpu.async_copy(src_hbm.at[s, pl.ds(blk * _K, _K)],
                         srcb.at[slot], isem0)
        pltpu.async_copy(dst_hbm.at[s, pl.ds(blk * _K, _K)],
                         dstb.at[slot], isem1)

    def wait_idx(blk, slot):
        pltpu.make_async_copy(src_hbm.at[s, pl.ds(blk * _K, _K)],
                              srcb.at[slot], isem0).wait()
        pltpu.make_async_copy(dst_hbm.at[s, pl.ds(blk * _K, _K)],
                              dstb.at[slot], isem1).wait()

    def fire_gather(slot, row, k):
        pltpu.async_copy(hme.at[srcb.at[slot, row]], bufs[k], gsem[k])

    def wait_gather(slot, row, k):
        pltpu.make_async_copy(hme.at[srcb.at[slot, row]], bufs[k],
                              gsem[k]).wait()

    def fire_scatter(slot, row, k):
        pltpu.async_copy(bufs[k], acc.at[dstb.at[slot, row]], ssem[k],
                         add=True)

    def wait_scatter(slot, row, k):
        pltpu.make_async_copy(bufs[k], acc.at[dstb.at[slot, row]],
                              ssem[k]).wait()

    def block(i, slot, first, last):
        nslot = 1 - slot
        for k in range(_K):
            if not last:
                if k == 0:
                    fire_idx(i + 1, nslot)
                if k == _S:
                    wait_idx(i + 1, nslot)
            if k < _S:
                if not first:
                    wait_scatter(nslot, k - _S + _K, (k - _S) % _K)
            else:
                wait_scatter(slot, k - _S, (k - _S) % _K)
            if k < _S:
                fire_gather(slot, k + _G, (k + _G) % _K)
            elif not last:
                fire_gather(nslot, k + _G - _K, (k + _G) % _K)
            wait_gather(slot, k, k)
            fire_scatter(slot, k, k)

    for k in range(_G):
        fire_gather(0, k, k)
    block(0, 0, True, False)

    def pair(bi, carry):
        block(2 * bi + 1, 1, False, False)
        block(2 * bi + 2, 0, False, False)
        return carry

    lax.fori_loop(0, (_NBLK - 2) // 2, pair, 0)

    block(_NBLK - 1, 1, False, True)
    for k in range(_S):
        wait_scatter(1, _G + k, (_G + k) % _K)

    plsc.subcore_barrier()
    pltpu.sync_copy(acc.at[pl.ds(s * _SR, _SR)],
                    out_hbm.at[c, pl.ds(s * _SR, _SR)])


_segsum = pl.kernel(
    _segsum_body,
    mesh=plsc.VectorSubcoreMesh(core_axis_name="c", subcore_axis_name="s"),
    compiler_params=pltpu.CompilerParams(use_tc_tiling_on_sc=False),
    out_type=jax.ShapeDtypeStruct((_NC, _ACC_R, _HH), jnp.float32),
    scratch_types=(
        [pltpu.VMEM((2, _K, _B), jnp.int32),
         pltpu.VMEM((2, _K, _B), jnp.int32),
         pltpu.VMEM_SHARED((_ACC_R, _HH), jnp.float32),
         pltpu.VMEM_SHARED((_ACC_R, _HH), jnp.float32)]
        + [pltpu.VMEM((_B, _HH), jnp.float32) for _ in range(_K)]
        + [pltpu.SemaphoreType.DMA for _ in range(2 * _K + 4)]),
)


def _pre_body(x_ref, g_ref, b_ref, w_ref, pb_ref, o_ref):
    x = x_ref[...]
    m = jnp.mean(x, axis=0, keepdims=True)
    d = x - m
    v = jnp.mean(d * d, axis=0, keepdims=True)
    hn = g_ref[...] * d * lax.rsqrt(v + 1e-5) + b_ref[...]
    w = w_ref[...]
    pb = pb_ref[...]
    o_ref[0] = jnp.maximum(
        jnp.dot(hn, w[:, :_HH], preferred_element_type=jnp.float32)
        + pb[:, :_HH], 0.0)
    o_ref[1] = jnp.maximum(
        jnp.dot(hn, w[:, _HH:], preferred_element_type=jnp.float32)
        + pb[:, _HH:], 0.0)


_pre = pl.pallas_call(
    _pre_body,
    out_shape=jax.ShapeDtypeStruct((2, _N, _HH), jnp.float32),
)


def _half_mlp_bn(z2, g, be):
    m = jnp.mean(z2, axis=0, keepdims=True)
    d = z2 - m
    v = jnp.mean(d * d, axis=0, keepdims=True)
    return g * d * lax.rsqrt(v + 1e-5) + be


def _conv_body(h_ref, a_ref, w1_ref, b1_ref, w2_ref, b2_ref,
               g_ref, be_ref, *rest, add_res):
    if add_res:
        r_ref, o_ref = rest
    else:
        (o_ref,) = rest
    z = jnp.concatenate([h_ref[0] + a_ref[0, :_N],
                         h_ref[1] + a_ref[1, :_N]], axis=1)
    u = jnp.maximum(
        jnp.dot(z, w1_ref[...], preferred_element_type=jnp.float32)
        + b1_ref[...], 0.0)
    z2 = jnp.dot(u, w2_ref[...], preferred_element_type=jnp.float32) \
        + b2_ref[...]
    h2 = jnp.maximum(_half_mlp_bn(z2, g_ref[...], be_ref[...]), 0.0)
    h2_0 = h2[:, :_HH]
    h2_1 = h2[:, _HH:]
    if add_res:
        h2_0 = h2_0 + r_ref[0]
        h2_1 = h2_1 + r_ref[1]
    o_ref[0] = h2_0
    o_ref[1] = h2_1


_conv = pl.pallas_call(
    functools.partial(_conv_body, add_res=False),
    out_shape=jax.ShapeDtypeStruct((2, _N, _HH), jnp.float32),
)

_conv_res = pl.pallas_call(
    functools.partial(_conv_body, add_res=True),
    out_shape=jax.ShapeDtypeStruct((2, _N, _HH), jnp.float32),
)


def _conv_post_body(h_ref, a_ref, w1_ref, b1_ref, w2_ref, b2_ref,
                    g_ref, be_ref, r_ref, w_ref, b_ref, emb_ref, log_ref):
    z = jnp.concatenate([h_ref[0] + a_ref[0, :_N],
                         h_ref[1] + a_ref[1, :_N]], axis=1)
    u = jnp.maximum(
        jnp.dot(z, w1_ref[...], preferred_element_type=jnp.float32)
        + b1_ref[...], 0.0)
    z2 = jnp.dot(u, w2_ref[...], preferred_element_type=jnp.float32) \
        + b2_ref[...]
    h2 = jnp.maximum(_half_mlp_bn(z2, g_ref[...], be_ref[...]), 0.0) \
        + jnp.concatenate([r_ref[0], r_ref[1]], axis=1)
    pooled = jnp.mean(h2, axis=0, keepdims=True)
    emb_ref[...] = pooled
    log_ref[...] = (
        jnp.dot(pooled, w_ref[...], preferred_element_type=jnp.float32)
        + b_ref[...])


_conv_post = pl.pallas_call(
    _conv_post_body,
    out_shape=(jax.ShapeDtypeStruct((1, _H), jnp.float32),
               jax.ShapeDtypeStruct((1, _NOTES), jnp.float32)),
)


def kernel(x, edge_index, fn_g, fn_b, proj_W, proj_b,
           W1_0, b1_0, W2_0, b2_0, g_0, be_0,
           W1_1, b1_1, W2_1, b2_1, g_1, be_1,
           W1_2, b1_2, W2_2, b2_2, g_2, be_2,
           pred_W, pred_b):
    src = edge_index[0]
    dst = edge_index[1]
    pad = _EPAD - _E
    src_p = jnp.concatenate(
        [src, jnp.zeros((pad,), jnp.int32)]).reshape(_NS, _NB, _B)
    dst_p = jnp.concatenate(
        [dst, jnp.full((pad,), _N, jnp.int32)]).reshape(_NS, _NB, _B)
    zeros = jnp.zeros((_ACC_R, _HH), jnp.float32)

    r2 = lambda t: t.reshape(1, -1)
    h = _pre(x, r2(fn_g), r2(fn_b), proj_W, r2(proj_b))
    res = h
    mlps = [(W1_0, r2(b1_0), W2_0, r2(b2_0), r2(g_0), r2(be_0)),
            (W1_1, r2(b1_1), W2_1, r2(b2_1), r2(g_1), r2(be_1)),
            (W1_2, r2(b1_2), W2_2, r2(b2_2), r2(g_2), r2(be_2))]
    for mi, (W1, b1, W2, b2, g, be) in enumerate(mlps):
        a = _segsum(h, src_p, dst_p, zeros)
        h = _conv(h, a, W1, b1, W2, b2, g, be)
        a = _segsum(h, src_p, dst_p, zeros)
        if mi < 2:
            h = _conv_res(h, a, W1, b1, W2, b2, g, be, res)
            res = h
        else:
            embed, logits = _conv_post(h, a, W1, b1, W2, b2, g, be, res,
                                       pred_W, r2(pred_b))
    return (embed, logits)
